# Initial kernel scaffold; baseline (speedup 1.0000x reference)
import jax
import jax.numpy as jnp
from jax.experimental import pallas as pl


def kernel(hidden_states, Wq, Wk, Wv, conv_q, conv_k, conv_v, W_init, ln_gamma, ln_beta, Wg, Wo):
    raise NotImplementedError("write your pallas kernel here")


# speedup vs baseline: 2.3611x; 2.3611x over previous
import jax
import jax.numpy as jnp
from jax.experimental import pallas as pl
from jax.experimental.pallas import tpu as pltpu

_KS = 4
_EPS_NORM = 1e-12
_EPS_LN = 1e-5
_H = 16
_DK = 64
_MC = 256


def _proj_body(x_ref, w_ref, o_ref):
    o_ref[0] = jnp.dot(x_ref[0], w_ref[...], preferred_element_type=jnp.float32)


def _shift_down(a, s):
    if s == 0:
        return a
    return jnp.concatenate([jnp.zeros((s, a.shape[1]), a.dtype), a[: a.shape[0] - s]], axis=0)


def _conv_silu(y, cw):
    acc = y * (1.0 + cw[_KS - 1 : _KS, :])
    for j in range(_KS - 1):
        acc = acc + _shift_down(y, _KS - 1 - j) * cw[j : j + 1, :]
    return acc * jax.nn.sigmoid(acc)


def _l2norm(y):
    return y * jax.lax.rsqrt(jnp.sum(y * y, axis=-1, keepdims=True) + _EPS_NORM)


def _rec_body(q_ref, k_ref, v_ref, w0_ref, cw_ref, gam_ref, bet_ref, o_ref):
    L = q_ref.shape[1]
    n_mc = L // _MC

    q = _l2norm(_conv_silu(q_ref[0], cw_ref[0, 0]))
    k = _l2norm(_conv_silu(k_ref[0], cw_ref[1, 0]))
    v = _conv_silu(v_ref[0], cw_ref[2, 0])

    rows = jax.lax.broadcasted_iota(jnp.int32, (_MC, _MC), 0)
    cols = jax.lax.broadcasted_iota(jnp.int32, (_MC, _MC), 1)
    causal = rows >= cols

    W = w0_ref[0, 0]
    gam = gam_ref[...]
    bet = bet_ref[...]
    for i in range(n_mc):
        sl = slice(i * _MC, (i + 1) * _MC)
        qq, kk, vv = q[sl], k[sl], v[sl]
        inter = jnp.dot(qq, W, preferred_element_type=jnp.float32)
        att = jax.lax.dot_general(qq, kk, (((1,), (1,)), ((), ())),
                                  preferred_element_type=jnp.float32)
        att = jnp.where(causal, att, 0.0)
        o = inter + jnp.dot(att, vv, preferred_element_type=jnp.float32)
        W = W + jax.lax.dot_general(kk, vv, (((0,), (0,)), ((), ())),
                                    preferred_element_type=jnp.float32)
        mu = jnp.mean(o, axis=-1, keepdims=True)
        d = o - mu
        var = jnp.mean(d * d, axis=-1, keepdims=True)
        o_ref[0, sl, :] = d * jax.lax.rsqrt(var + _EPS_LN) * gam + bet


def _out_body(o_ref, x_ref, wg_ref, wo_ref, out_ref):
    gate = jnp.dot(x_ref[0], wg_ref[...], preferred_element_type=jnp.float32)
    out_ref[0] = jnp.dot(o_ref[0] * gate, wo_ref[...],
                         preferred_element_type=jnp.float32)


def kernel(hidden_states, Wq, Wk, Wv, conv_q, conv_k, conv_v,
           W_init, ln_gamma, ln_beta, Wg, Wo):
    b, l, dim = hidden_states.shape
    h, dk = _H, _DK
    bh = b * h

    W3 = jnp.concatenate([Wq, Wk, Wv], axis=0).T
    TL = 256
    nt = l // TL
    qkv = pl.pallas_call(
        _proj_body,
        grid=(b, nt),
        in_specs=[
            pl.BlockSpec((1, TL, dim), lambda i, j: (i, j, 0)),
            pl.BlockSpec((dim, 3 * dim), lambda i, j: (0, 0)),
        ],
        out_specs=pl.BlockSpec((1, TL, 3 * dim), lambda i, j: (i, j, 0)),
        out_shape=jax.ShapeDtypeStruct((b, l, 3 * dim), jnp.float32),
        compiler_params=pltpu.CompilerParams(
            dimension_semantics=("parallel", "parallel"),
            vmem_limit_bytes=56 * 1024 * 1024,
        ),
        name="titans_qkv_proj",
    )(hidden_states, W3)

    qkvh = (qkv.reshape(b, l, 3, h, dk)
               .transpose(2, 0, 3, 1, 4)
               .reshape(3 * bh, l, dk))

    cw = (jnp.stack([conv_q, conv_k, conv_v], axis=0)
             .reshape(3, h, dk, _KS)
             .transpose(0, 1, 3, 2))
    gam = ln_gamma.reshape(1, dk)
    bet = ln_beta.reshape(1, dk)

    o_h = pl.pallas_call(
        _rec_body,
        grid=(bh,),
        in_specs=[
            pl.BlockSpec((1, l, dk), lambda c: (c, 0, 0)),
            pl.BlockSpec((1, l, dk), lambda c: (bh + c, 0, 0)),
            pl.BlockSpec((1, l, dk), lambda c: (2 * bh + c, 0, 0)),
            pl.BlockSpec((1, 1, dk, dk), lambda c: (0, c % _H, 0, 0)),
            pl.BlockSpec((3, 1, _KS, dk), lambda c: (0, c % _H, 0, 0)),
            pl.BlockSpec((1, dk), lambda c: (0, 0)),
            pl.BlockSpec((1, dk), lambda c: (0, 0)),
        ],
        out_specs=pl.BlockSpec((1, l, dk), lambda c: (c, 0, 0)),
        out_shape=jax.ShapeDtypeStruct((bh, l, dk), jnp.float32),
        compiler_params=pltpu.CompilerParams(
            dimension_semantics=("parallel",),
            vmem_limit_bytes=56 * 1024 * 1024,
        ),
        name="titans_recurrence",
    )(qkvh, qkvh, qkvh, W_init, cw, gam, bet)

    o = o_h.reshape(b, h, l, dk).transpose(0, 2, 1, 3).reshape(b, l, dim)

    out = pl.pallas_call(
        _out_body,
        grid=(b, nt),
        in_specs=[
            pl.BlockSpec((1, TL, dim), lambda i, j: (i, j, 0)),
            pl.BlockSpec((1, TL, dim), lambda i, j: (i, j, 0)),
            pl.BlockSpec((dim, dim), lambda i, j: (0, 0)),
            pl.BlockSpec((dim, dim), lambda i, j: (0, 0)),
        ],
        out_specs=pl.BlockSpec((1, TL, dim), lambda i, j: (i, j, 0)),
        out_shape=jax.ShapeDtypeStruct((b, l, dim), jnp.float32),
        compiler_params=pltpu.CompilerParams(
            dimension_semantics=("parallel", "parallel"),
            vmem_limit_bytes=56 * 1024 * 1024,
        ),
        name="titans_out_proj",
    )(o, hidden_states, Wg.T, Wo.T)
    return out


# speedup vs baseline: 3.1253x; 1.3237x over previous
import jax
import jax.numpy as jnp
from jax.experimental import pallas as pl
from jax.experimental.pallas import tpu as pltpu

_KS = 4
_EPS_NORM = 1e-12
_EPS_LN = 1e-5
_H = 16
_DK = 64
_MC = 256
_TL = 256

_SEM_PROJ = ("parallel", "parallel")
_SEM_REC = ("parallel",)


def _proj_body(x_ref, w_ref, o_ref):
    o_ref[0] = jnp.dot(x_ref[0], w_ref[...], preferred_element_type=jnp.float32)


def _shift_down(a, s):
    return jnp.concatenate(
        [jnp.zeros((s, a.shape[1]), a.dtype), a[: a.shape[0] - s]], axis=0)


def _conv_silu(y, cw):
    acc = y * (1.0 + cw[_KS - 1 : _KS, :])
    for j in range(_KS - 1):
        acc = acc + _shift_down(y, _KS - 1 - j) * cw[j : j + 1, :]
    return acc * jax.nn.sigmoid(acc)


def _rec_body(q_ref, k_ref, v_ref, g_ref, w0_ref, cw_ref, gam_ref, bet_ref,
              o_ref):
    L = q_ref.shape[1]
    n_mc = L // _MC

    q = _conv_silu(q_ref[0], cw_ref[0, 0])
    k = _conv_silu(k_ref[0], cw_ref[1, 0])
    v = _conv_silu(v_ref[0], cw_ref[2, 0])

    lane = jax.lax.broadcasted_iota(jnp.int32, (1, 128), 1)
    left = lane < _DK

    def _headnorm(y):
        y2 = y * y
        sl = jnp.sum(y2[:, :_DK], axis=-1, keepdims=True)
        sr = jnp.sum(y2[:, _DK:], axis=-1, keepdims=True)
        return y * jnp.where(left, jax.lax.rsqrt(sl + _EPS_NORM),
                             jax.lax.rsqrt(sr + _EPS_NORM))

    q = _headnorm(q)
    k = _headnorm(k)

    rows = jax.lax.broadcasted_iota(jnp.int32, (_MC, _MC), 0)
    cols = jax.lax.broadcasted_iota(jnp.int32, (_MC, _MC), 1)
    causal = rows >= cols
    r128 = jax.lax.broadcasted_iota(jnp.int32, (128, 128), 0)
    c128 = jax.lax.broadcasted_iota(jnp.int32, (128, 128), 1)
    blockdiag = (r128 // _DK) == (c128 // _DK)

    W = w0_ref[0]
    gam = gam_ref[...]
    bet = bet_ref[...]
    inv_dk = 1.0 / _DK
    for i in range(n_mc):
        sl_ = slice(i * _MC, (i + 1) * _MC)
        qq, kk, vv = q[sl_], k[sl_], v[sl_]
        inter = jnp.dot(qq, W, preferred_element_type=jnp.float32)
        aa = jax.lax.dot_general(qq[:, :_DK], kk[:, :_DK],
                                 (((1,), (1,)), ((), ())),
                                 preferred_element_type=jnp.float32)
        ab = jax.lax.dot_general(qq[:, _DK:], kk[:, _DK:],
                                 (((1,), (1,)), ((), ())),
                                 preferred_element_type=jnp.float32)
        aa = jnp.where(causal, aa, 0.0)
        ab = jnp.where(causal, ab, 0.0)
        intra = (jnp.dot(aa, jnp.where(left, vv, 0.0),
                         preferred_element_type=jnp.float32)
                 + jnp.dot(ab, jnp.where(left, 0.0, vv),
                           preferred_element_type=jnp.float32))
        o = inter + intra
        upd = jax.lax.dot_general(kk, vv, (((0,), (0,)), ((), ())),
                                  preferred_element_type=jnp.float32)
        W = W + jnp.where(blockdiag, upd, 0.0)
        mul_ = jnp.sum(o[:, :_DK], axis=-1, keepdims=True) * inv_dk
        mur = jnp.sum(o[:, _DK:], axis=-1, keepdims=True) * inv_dk
        d = o - jnp.where(left, mul_, mur)
        d2 = d * d
        vl = jnp.sum(d2[:, :_DK], axis=-1, keepdims=True) * inv_dk
        vr = jnp.sum(d2[:, _DK:], axis=-1, keepdims=True) * inv_dk
        o_ln = d * jax.lax.rsqrt(jnp.where(left, vl, vr) + _EPS_LN) * gam + bet
        o_ref[0, sl_, :] = o_ln * g_ref[0, sl_, :]


def _out_body(og_ref, wo_ref, out_ref):
    out_ref[0] = jnp.dot(og_ref[0], wo_ref[...],
                         preferred_element_type=jnp.float32)


def kernel(hidden_states, Wq, Wk, Wv, conv_q, conv_k, conv_v,
           W_init, ln_gamma, ln_beta, Wg, Wo):
    b, l, dim = hidden_states.shape
    h, dk = _H, _DK
    npair = h // 2

    W4 = jnp.concatenate([Wq, Wk, Wv, Wg], axis=0).T
    nt = l // _TL
    y4 = pl.pallas_call(
        _proj_body,
        grid=(b, nt),
        in_specs=[
            pl.BlockSpec((1, _TL, dim), lambda i, j: (i, j, 0)),
            pl.BlockSpec((dim, 4 * dim), lambda i, j: (0, 0)),
        ],
        out_specs=pl.BlockSpec((1, _TL, 4 * dim), lambda i, j: (i, j, 0)),
        out_shape=jax.ShapeDtypeStruct((b, l, 4 * dim), jnp.float32),
        compiler_params=pltpu.CompilerParams(
            dimension_semantics=_SEM_PROJ,
            vmem_limit_bytes=56 * 1024 * 1024,
        ),
        name="titans_qkvg_proj",
    )(hidden_states, W4)

    wp = W_init[0].reshape(npair, 2, dk, dk)
    w0 = jnp.zeros((npair, 2, dk, 2, dk), jnp.float32)
    w0 = w0.at[:, 0, :, 0, :].set(wp[:, 0]).at[:, 1, :, 1, :].set(wp[:, 1])
    w0 = w0.reshape(npair, 2 * dk, 2 * dk)

    cw = (jnp.stack([conv_q, conv_k, conv_v], axis=0)
             .reshape(3, npair, 2 * dk, _KS)
             .transpose(0, 1, 3, 2))
    gam = jnp.tile(ln_gamma, 2).reshape(1, 2 * dk)
    bet = jnp.tile(ln_beta, 2).reshape(1, 2 * dk)

    og = pl.pallas_call(
        _rec_body,
        grid=(b * npair,),
        in_specs=[
            pl.BlockSpec((1, l, 2 * dk), lambda c: (c // npair, 0, c % npair)),
            pl.BlockSpec((1, l, 2 * dk),
                         lambda c: (c // npair, 0, npair + c % npair)),
            pl.BlockSpec((1, l, 2 * dk),
                         lambda c: (c // npair, 0, 2 * npair + c % npair)),
            pl.BlockSpec((1, l, 2 * dk),
                         lambda c: (c // npair, 0, 3 * npair + c % npair)),
            pl.BlockSpec((1, 2 * dk, 2 * dk), lambda c: (c % npair, 0, 0)),
            pl.BlockSpec((3, 1, _KS, 2 * dk), lambda c: (0, c % npair, 0, 0)),
            pl.BlockSpec((1, 2 * dk), lambda c: (0, 0)),
            pl.BlockSpec((1, 2 * dk), lambda c: (0, 0)),
        ],
        out_specs=pl.BlockSpec((1, l, 2 * dk), lambda c: (c // npair, 0, c % npair)),
        out_shape=jax.ShapeDtypeStruct((b, l, dim), jnp.float32),
        compiler_params=pltpu.CompilerParams(
            dimension_semantics=_SEM_REC,
            vmem_limit_bytes=56 * 1024 * 1024,
        ),
        name="titans_recurrence",
    )(y4, y4, y4, y4, w0, cw, gam, bet)

    out = pl.pallas_call(
        _out_body,
        grid=(b, nt),
        in_specs=[
            pl.BlockSpec((1, _TL, dim), lambda i, j: (i, j, 0)),
            pl.BlockSpec((dim, dim), lambda i, j: (0, 0)),
        ],
        out_specs=pl.BlockSpec((1, _TL, dim), lambda i, j: (i, j, 0)),
        out_shape=jax.ShapeDtypeStruct((b, l, dim), jnp.float32),
        compiler_params=pltpu.CompilerParams(
            dimension_semantics=_SEM_PROJ,
            vmem_limit_bytes=56 * 1024 * 1024,
        ),
        name="titans_out_proj",
    )(og, Wo.T)
    return out


# speedup vs baseline: 3.9426x; 1.2615x over previous
import jax
import jax.numpy as jnp
from jax.experimental import pallas as pl
from jax.experimental.pallas import tpu as pltpu

_KS = 4
_EPS_NORM = 1e-12
_EPS_LN = 1e-5
_H = 16
_DK = 64
_MC = 256
_TL = 256

_SEM_PROJ = ("parallel", "parallel")
_SEM_REC = ("parallel",)
_VMEM = 56 * 1024 * 1024


def _proj_body(x_ref, xp_ref, w_ref, cw_ref, g_ref, gt_ref, o_ref):
    j = pl.program_id(1)
    x = x_ref[0]
    y = jnp.dot(x, w_ref[...], preferred_element_type=jnp.float32)
    xh = xp_ref[0, _TL - _KS + 1 :, :]
    yh = jnp.dot(xh, w_ref[...], preferred_element_type=jnp.float32)[:, :3072]
    yh = yh * jnp.where(j > 0, 1.0, 0.0)

    z = y[:, :3072]
    acc = z * (1.0 + cw_ref[_KS - 1 : _KS, :])
    for s in range(1, _KS):
        zs = jnp.concatenate([yh[_KS - 1 - s :], z[: _TL - s]], axis=0)
        acc = acc + zs * cw_ref[_KS - 1 - s : _KS - s, :]
    z = acc * jax.nn.sigmoid(acc)

    zqk = z[:, :2048]
    s32 = jnp.dot(zqk * zqk, g_ref[...], preferred_element_type=jnp.float32)
    scale = jnp.dot(jax.lax.rsqrt(s32 + _EPS_NORM), gt_ref[...],
                    preferred_element_type=jnp.float32)
    o_ref[0, :, :2048] = zqk * scale
    o_ref[0, :, 2048:3072] = z[:, 2048:]
    o_ref[0, :, 3072:] = y[:, 3072:]


def _rec_body(q_ref, k_ref, v_ref, w0_ref, o_ref):
    L = q_ref.shape[1]
    n_mc = L // _MC

    q = q_ref[0]
    k = k_ref[0]
    v = v_ref[0]

    lane = jax.lax.broadcasted_iota(jnp.int32, (1, 128), 1)
    left = lane < _DK
    rows = jax.lax.broadcasted_iota(jnp.int32, (_MC, _MC), 0)
    cols = jax.lax.broadcasted_iota(jnp.int32, (_MC, _MC), 1)
    causal = rows >= cols
    r128 = jax.lax.broadcasted_iota(jnp.int32, (128, 128), 0)
    c128 = jax.lax.broadcasted_iota(jnp.int32, (128, 128), 1)
    blockdiag = (r128 // _DK) == (c128 // _DK)

    W = w0_ref[0]
    for i in range(n_mc):
        sl_ = slice(i * _MC, (i + 1) * _MC)
        qq, kk, vv = q[sl_], k[sl_], v[sl_]
        inter = jnp.dot(qq, W, preferred_element_type=jnp.float32)
        aa = jax.lax.dot_general(qq[:, :_DK], kk[:, :_DK],
                                 (((1,), (1,)), ((), ())),
                                 preferred_element_type=jnp.float32)
        ab = jax.lax.dot_general(qq[:, _DK:], kk[:, _DK:],
                                 (((1,), (1,)), ((), ())),
                                 preferred_element_type=jnp.float32)
        aa = jnp.where(causal, aa, 0.0)
        ab = jnp.where(causal, ab, 0.0)
        intra = (jnp.dot(aa, jnp.where(left, vv, 0.0),
                         preferred_element_type=jnp.float32)
                 + jnp.dot(ab, jnp.where(left, 0.0, vv),
                           preferred_element_type=jnp.float32))
        upd = jax.lax.dot_general(kk, vv, (((0,), (0,)), ((), ())),
                                  preferred_element_type=jnp.float32)
        o_ref[0, sl_, :] = inter + intra
        W = W + jnp.where(blockdiag, upd, 0.0)


def _out_body(o_ref, g_ref, g64_ref, g64t_ref, gam_ref, bet_ref, wo_ref,
              out_ref):
    o = o_ref[0]
    inv = 1.0 / _DK
    mu = jnp.dot(jnp.dot(o, g64_ref[...], preferred_element_type=jnp.float32)
                 * inv, g64t_ref[...], preferred_element_type=jnp.float32)
    d = o - mu
    v16 = jnp.dot(d * d, g64_ref[...], preferred_element_type=jnp.float32) * inv
    scale = jnp.dot(jax.lax.rsqrt(v16 + _EPS_LN), g64t_ref[...],
                    preferred_element_type=jnp.float32)
    o_ln = d * scale * gam_ref[...] + bet_ref[...]
    out_ref[0] = jnp.dot(o_ln * g_ref[0], wo_ref[...],
                         preferred_element_type=jnp.float32)


def kernel(hidden_states, Wq, Wk, Wv, conv_q, conv_k, conv_v,
           W_init, ln_gamma, ln_beta, Wg, Wo):
    b, l, dim = hidden_states.shape
    h, dk = _H, _DK
    npair = h // 2
    nt = l // _TL

    W4 = jnp.concatenate([Wq, Wk, Wv, Wg], axis=0).T
    cw3 = jnp.concatenate([conv_q, conv_k, conv_v], axis=0).T
    eye32 = jnp.eye(2 * h, dtype=jnp.float32)
    G = jnp.repeat(eye32, dk, axis=0)
    GT = G.T
    eye16 = jnp.eye(h, dtype=jnp.float32)
    G64 = jnp.repeat(eye16, dk, axis=0)
    G64T = G64.T
    gam = jnp.tile(ln_gamma, h).reshape(1, dim)
    bet = jnp.tile(ln_beta, h).reshape(1, dim)

    wp = W_init[0].reshape(npair, 2, dk, dk)
    w0 = jnp.zeros((npair, 2, dk, 2, dk), jnp.float32)
    w0 = w0.at[:, 0, :, 0, :].set(wp[:, 0]).at[:, 1, :, 1, :].set(wp[:, 1])
    w0 = w0.reshape(npair, 2 * dk, 2 * dk)

    y4 = pl.pallas_call(
        _proj_body,
        grid=(b, nt),
        in_specs=[
            pl.BlockSpec((1, _TL, dim), lambda i, j: (i, j, 0)),
            pl.BlockSpec((1, _TL, dim),
                         lambda i, j: (i, jnp.maximum(j - 1, 0), 0)),
            pl.BlockSpec((dim, 4 * dim), lambda i, j: (0, 0)),
            pl.BlockSpec((_KS, 3 * dim), lambda i, j: (0, 0)),
            pl.BlockSpec((2 * dim, 2 * h), lambda i, j: (0, 0)),
            pl.BlockSpec((2 * h, 2 * dim), lambda i, j: (0, 0)),
        ],
        out_specs=pl.BlockSpec((1, _TL, 4 * dim), lambda i, j: (i, j, 0)),
        out_shape=jax.ShapeDtypeStruct((b, l, 4 * dim), jnp.float32),
        compiler_params=pltpu.CompilerParams(
            dimension_semantics=_SEM_PROJ,
            vmem_limit_bytes=_VMEM,
        ),
        name="titans_qkvg_proj",
    )(hidden_states, hidden_states, W4, cw3, G, GT)

    o_pre = pl.pallas_call(
        _rec_body,
        grid=(b * npair,),
        in_specs=[
            pl.BlockSpec((1, l, 2 * dk), lambda c: (c // npair, 0, c % npair)),
            pl.BlockSpec((1, l, 2 * dk),
                         lambda c: (c // npair, 0, npair + c % npair)),
            pl.BlockSpec((1, l, 2 * dk),
                         lambda c: (c // npair, 0, 2 * npair + c % npair)),
            pl.BlockSpec((1, 2 * dk, 2 * dk), lambda c: (c % npair, 0, 0)),
        ],
        out_specs=pl.BlockSpec((1, l, 2 * dk),
                               lambda c: (c // npair, 0, c % npair)),
        out_shape=jax.ShapeDtypeStruct((b, l, dim), jnp.float32),
        compiler_params=pltpu.CompilerParams(
            dimension_semantics=_SEM_REC,
            vmem_limit_bytes=_VMEM,
        ),
        name="titans_recurrence",
    )(y4, y4, y4, w0)

    out = pl.pallas_call(
        _out_body,
        grid=(b, nt),
        in_specs=[
            pl.BlockSpec((1, _TL, dim), lambda i, j: (i, j, 0)),
            pl.BlockSpec((1, _TL, dim), lambda i, j: (i, j, 3)),
            pl.BlockSpec((dim, h), lambda i, j: (0, 0)),
            pl.BlockSpec((h, dim), lambda i, j: (0, 0)),
            pl.BlockSpec((1, dim), lambda i, j: (0, 0)),
            pl.BlockSpec((1, dim), lambda i, j: (0, 0)),
            pl.BlockSpec((dim, dim), lambda i, j: (0, 0)),
        ],
        out_specs=pl.BlockSpec((1, _TL, dim), lambda i, j: (i, j, 0)),
        out_shape=jax.ShapeDtypeStruct((b, l, dim), jnp.float32),
        compiler_params=pltpu.CompilerParams(
            dimension_semantics=_SEM_PROJ,
            vmem_limit_bytes=_VMEM,
        ),
        name="titans_out_proj",
    )(o_pre, y4, G64, G64T, gam, bet, Wo.T)
    return out
